# baseline (speedup 1.0000x reference)
import functools

import jax
import jax.numpy as jnp
from jax import lax
from jax.experimental import pallas as pl
from jax.experimental.pallas import tpu as pltpu
from jax.experimental.pallas import tpu_sc as plsc

EMBED_DIM = 64
MODEL_DIM = 128
_CH = 80
_LANES = 16
_REP_D = 16
_REP_P = 8


def _tc_preproject(tbl, w, bias=None, block_rows=None, replicas=1):
    n = tbl.shape[0]
    br = block_rows or n

    if bias is None:
        def body(t_ref, w_ref, o_ref):
            o_ref[...] = jnp.dot(t_ref[...], w_ref[...],
                                 preferred_element_type=jnp.float32)
        extra_in, extra_spec = (), ()
    else:
        def body(t_ref, w_ref, b_ref, o_ref):
            o_ref[...] = jnp.dot(t_ref[...], w_ref[...],
                                 preferred_element_type=jnp.float32) + b_ref[...]
        extra_in = (bias.reshape(1, MODEL_DIM),)
        extra_spec = (pl.BlockSpec((1, MODEL_DIM), lambda i: (0, 0)),)

    nblocks = n // br
    return pl.pallas_call(
        body,
        grid=(replicas * nblocks,),
        in_specs=[
            pl.BlockSpec((br, EMBED_DIM), lambda i: (i % nblocks, 0)),
            pl.BlockSpec((EMBED_DIM, MODEL_DIM), lambda i: (0, 0)),
            *extra_spec,
        ],
        out_specs=pl.BlockSpec((br, MODEL_DIM), lambda i: (i, 0)),
        out_shape=jax.ShapeDtypeStruct((replicas * n, MODEL_DIM), jnp.float32),
    )(tbl, w, *extra_in)


def _tc_prep_all(wd, wi, wp, w_proj, b_proj, block_rows):
    n_img = wi.shape[0]
    nblocks = n_img // block_rows
    nd, np_ = wd.shape[0], wp.shape[0]
    e = EMBED_DIM

    def body(wi_ref, wd_ref, wp_ref, w_ref, b_ref, pi_ref, pd_ref, pp_ref):
        pi_ref[...] = jnp.dot(wi_ref[...], w_ref[e:2 * e, :],
                              preferred_element_type=jnp.float32)
        pd_ref[...] = jnp.dot(wd_ref[...], w_ref[0:e, :],
                              preferred_element_type=jnp.float32) + b_ref[...]
        pp_ref[...] = jnp.dot(wp_ref[...], w_ref[2 * e:3 * e, :],
                              preferred_element_type=jnp.float32)

    return pl.pallas_call(
        body,
        grid=(nblocks,),
        in_specs=[
            pl.BlockSpec((block_rows, e), lambda i: (i, 0)),
            pl.BlockSpec((nd, e), lambda i: (0, 0)),
            pl.BlockSpec((np_, e), lambda i: (0, 0)),
            pl.BlockSpec((3 * e, MODEL_DIM), lambda i: (0, 0)),
            pl.BlockSpec((1, MODEL_DIM), lambda i: (0, 0)),
        ],
        out_specs=[
            pl.BlockSpec((block_rows, MODEL_DIM), lambda i: (i, 0)),
            pl.BlockSpec((nd, MODEL_DIM), lambda i: (i % _REP_D, 0)),
            pl.BlockSpec((np_, MODEL_DIM), lambda i: (i % _REP_P, 0)),
        ],
        out_shape=[
            jax.ShapeDtypeStruct((n_img, MODEL_DIM), jnp.float32),
            jax.ShapeDtypeStruct((_REP_D * nd, MODEL_DIM), jnp.float32),
            jax.ShapeDtypeStruct((_REP_P * np_, MODEL_DIM), jnp.float32),
        ],
    )(wi, wd, wp, w_proj, b_proj.reshape(1, MODEL_DIM))


def _sc_gather_sum(did, iid, pid, pd, pp, pi):
    info = plsc.get_sparse_core_info()
    nc, ns = info.num_cores, info.num_subcores
    nw = nc * ns
    tok = did.shape[0]
    per_w = tok // nw
    idb = 3200
    nblk = per_w // idb
    nch = idb // _CH
    nd, np_ = pd.shape[0], pp.shape[0]
    nd_base, np_base = nd // _REP_D, np_ // _REP_P

    @functools.partial(
        pl.kernel,
        mesh=plsc.VectorSubcoreMesh(core_axis_name="c", subcore_axis_name="s"),
        out_type=jax.ShapeDtypeStruct((tok, MODEL_DIM), jnp.float32),
        scratch_types=[
            pltpu.VMEM((idb,), jnp.int32),
            pltpu.VMEM((idb,), jnp.int32),
            pltpu.VMEM((idb,), jnp.int32),
            [pltpu.VMEM((_CH, MODEL_DIM), jnp.float32) for _ in range(2)],
            [pltpu.VMEM((_CH, MODEL_DIM), jnp.float32) for _ in range(2)],
            [pltpu.VMEM((_CH, MODEL_DIM), jnp.float32) for _ in range(2)],
            [pltpu.VMEM((_CH, MODEL_DIM), jnp.float32) for _ in range(2)],
            [pltpu.SemaphoreType.DMA for _ in range(2)],
            [pltpu.SemaphoreType.DMA for _ in range(2)],
        ],
    )
    def k(did_h, iid_h, pid_h, pd_h, pp_h, pi_h, out_h,
          xd, xi, xp, gd, gi, gp, ob, gsem, osem):
        sid = lax.axis_index("s")
        wid = sid * nc + lax.axis_index("c")
        base = wid * per_w

        if _REP_D > 1 or _REP_P > 1:
            lane = lax.iota(jnp.int32, _LANES)
            spread_d = (lane % _REP_D) * nd_base
            spread_p = (lane % _REP_P) * np_base

        def fire(ch, par):
            off = ch * _CH
            pltpu.async_copy(pi_h.at[xi.at[pl.ds(off, _CH)]], gi[par], gsem[par])
            pltpu.async_copy(pd_h.at[xd.at[pl.ds(off, _CH)]], gd[par], gsem[par])
            pltpu.async_copy(pp_h.at[xp.at[pl.ds(off, _CH)]], gp[par], gsem[par])

        def gwait(par):
            pltpu.make_async_copy(pi_h.at[pl.ds(0, _CH)], gi[par], gsem[par]).wait()
            pltpu.make_async_copy(pi_h.at[pl.ds(0, _CH)], gd[par], gsem[par]).wait()
            pltpu.make_async_copy(pi_h.at[pl.ds(0, _CH)], gp[par], gsem[par]).wait()

        def owait(par):
            pltpu.make_async_copy(pi_h.at[pl.ds(0, _CH)], ob[par], osem[par]).wait()

        def blk(bi_, carry):
            boff = base + bi_ * idb
            pltpu.sync_copy(did_h.at[pl.ds(boff, idb)], xd)
            pltpu.sync_copy(iid_h.at[pl.ds(boff, idb)], xi)
            pltpu.sync_copy(pid_h.at[pl.ds(boff, idb)], xp)

            if _REP_D > 1 or _REP_P > 1:
                def spread(j, c0):
                    s = pl.ds(j * _LANES, _LANES)
                    xd[s] = xd[s] + spread_d
                    xp[s] = xp[s] + spread_p
                    return c0

                lax.fori_loop(0, idb // _LANES, spread, 0)

            fire(0, 0)
            fire(1, 1)

            def pair(pr, c1):
                for par in (0, 1):
                    ch = pr * 2 + par
                    g = bi_ * nch + ch
                    gwait(par)

                    @pl.when(g >= 2)
                    def _drain_prev_write():
                        owait(par)

                    def row(j, c2):
                        for kk in range(MODEL_DIM // _LANES):
                            s = pl.ds(kk * _LANES, _LANES)
                            ob[par][j, s] = gi[par][j, s] + gd[par][j, s] + gp[par][j, s]
                        return c2

                    lax.fori_loop(0, _CH, row, 0)
                    pltpu.async_copy(ob[par], out_h.at[pl.ds(boff + ch * _CH, _CH)],
                                     osem[par])

                    @pl.when(ch + 2 < nch)
                    def _prefetch_next():
                        fire(ch + 2, par)
                return c1

            lax.fori_loop(0, nch // 2, pair, 0)
            return carry

        lax.fori_loop(0, nblk, blk, 0)
        owait(0)
        owait(1)

    return k(did, iid, pid, pd, pp, pi)


def kernel(dataset_ids, image_ids, patch_ids, W_dataset, W_image, W_patch,
           W_proj, b_proj):
    b, l = dataset_ids.shape
    did = dataset_ids.reshape(-1).astype(jnp.int32)
    iid = image_ids.reshape(-1).astype(jnp.int32)
    pid = patch_ids.reshape(-1).astype(jnp.int32)
    pi, pd, pp = _tc_prep_all(W_dataset, W_image, W_patch, W_proj, b_proj,
                              block_rows=20000)
    out = _sc_gather_sum(did, iid, pid, pd, pp, pi)
    return out.reshape(b, l, MODEL_DIM)


# speedup vs baseline: 1.0250x; 1.0250x over previous
import functools

import jax
import jax.numpy as jnp
from jax import lax
from jax.experimental import pallas as pl
from jax.experimental.pallas import tpu as pltpu
from jax.experimental.pallas import tpu_sc as plsc

EMBED_DIM = 64
MODEL_DIM = 128
_CH = 80
_LANES = 16
_REP_D = 16
_REP_P = 8


def _tc_preproject(tbl, w, bias=None, block_rows=None, replicas=1):
    n = tbl.shape[0]
    br = block_rows or n

    if bias is None:
        def body(t_ref, w_ref, o_ref):
            o_ref[...] = jnp.dot(t_ref[...], w_ref[...],
                                 preferred_element_type=jnp.float32)
        extra_in, extra_spec = (), ()
    else:
        def body(t_ref, w_ref, b_ref, o_ref):
            o_ref[...] = jnp.dot(t_ref[...], w_ref[...],
                                 preferred_element_type=jnp.float32) + b_ref[...]
        extra_in = (bias.reshape(1, MODEL_DIM),)
        extra_spec = (pl.BlockSpec((1, MODEL_DIM), lambda i: (0, 0)),)

    nblocks = n // br
    return pl.pallas_call(
        body,
        grid=(replicas * nblocks,),
        in_specs=[
            pl.BlockSpec((br, EMBED_DIM), lambda i: (i % nblocks, 0)),
            pl.BlockSpec((EMBED_DIM, MODEL_DIM), lambda i: (0, 0)),
            *extra_spec,
        ],
        out_specs=pl.BlockSpec((br, MODEL_DIM), lambda i: (i, 0)),
        out_shape=jax.ShapeDtypeStruct((replicas * n, MODEL_DIM), jnp.float32),
    )(tbl, w, *extra_in)


def _sc_gather_sum(did, iid, pid, pd, pp, pi):
    info = plsc.get_sparse_core_info()
    nc, ns = info.num_cores, info.num_subcores
    nw = nc * ns
    tok = did.shape[0]
    per_w = tok // nw
    idb = 6400
    nblk = per_w // idb
    nch = idb // _CH
    nd, np_ = pd.shape[0], pp.shape[0]
    nd_base, np_base = nd // _REP_D, np_ // _REP_P

    @functools.partial(
        pl.kernel,
        mesh=plsc.VectorSubcoreMesh(core_axis_name="c", subcore_axis_name="s"),
        out_type=jax.ShapeDtypeStruct((tok, MODEL_DIM), jnp.float32),
        scratch_types=[
            pltpu.VMEM((idb,), jnp.int32),
            pltpu.VMEM((idb,), jnp.int32),
            pltpu.VMEM((idb,), jnp.int32),
            [pltpu.VMEM((_CH, MODEL_DIM), jnp.float32) for _ in range(2)],
            [pltpu.VMEM((_CH, MODEL_DIM), jnp.float32) for _ in range(2)],
            [pltpu.VMEM((_CH, MODEL_DIM), jnp.float32) for _ in range(2)],
            [pltpu.VMEM((_CH, MODEL_DIM), jnp.float32) for _ in range(2)],
            [pltpu.SemaphoreType.DMA for _ in range(2)],
            [pltpu.SemaphoreType.DMA for _ in range(2)],
        ],
    )
    def k(did_h, iid_h, pid_h, pd_h, pp_h, pi_h, out_h,
          xd, xi, xp, gd, gi, gp, ob, gsem, osem):
        sid = lax.axis_index("s")
        wid = sid * nc + lax.axis_index("c")
        base = wid * per_w

        if _REP_D > 1 or _REP_P > 1:
            lane = lax.iota(jnp.int32, _LANES)
            spread_d = (lane % _REP_D) * nd_base
            spread_p = (lane % _REP_P) * np_base

        def fire(ch, par):
            off = ch * _CH
            pltpu.async_copy(pi_h.at[xi.at[pl.ds(off, _CH)]], gi[par], gsem[par])
            pltpu.async_copy(pd_h.at[xd.at[pl.ds(off, _CH)]], gd[par], gsem[par])
            pltpu.async_copy(pp_h.at[xp.at[pl.ds(off, _CH)]], gp[par], gsem[par])

        def gwait(par):
            pltpu.make_async_copy(pi_h.at[pl.ds(0, _CH)], gi[par], gsem[par]).wait()
            pltpu.make_async_copy(pi_h.at[pl.ds(0, _CH)], gd[par], gsem[par]).wait()
            pltpu.make_async_copy(pi_h.at[pl.ds(0, _CH)], gp[par], gsem[par]).wait()

        def owait(par):
            pltpu.make_async_copy(pi_h.at[pl.ds(0, _CH)], ob[par], osem[par]).wait()

        def blk(bi_, carry):
            boff = base + bi_ * idb
            pltpu.sync_copy(did_h.at[pl.ds(boff, idb)], xd)
            pltpu.sync_copy(iid_h.at[pl.ds(boff, idb)], xi)
            pltpu.sync_copy(pid_h.at[pl.ds(boff, idb)], xp)

            if _REP_D > 1 or _REP_P > 1:
                def spread(j, c0):
                    s = pl.ds(j * _LANES, _LANES)
                    xd[s] = xd[s] + spread_d
                    xp[s] = xp[s] + spread_p
                    return c0

                lax.fori_loop(0, idb // _LANES, spread, 0)

            fire(0, 0)
            fire(1, 1)

            def pair(pr, c1):
                for par in (0, 1):
                    ch = pr * 2 + par
                    g = bi_ * nch + ch
                    gwait(par)

                    @pl.when(g >= 2)
                    def _drain_prev_write():
                        owait(par)

                    def row(j, c2):
                        for kk in range(MODEL_DIM // _LANES):
                            s = pl.ds(kk * _LANES, _LANES)
                            ob[par][j, s] = gi[par][j, s] + gd[par][j, s] + gp[par][j, s]
                        return c2

                    lax.fori_loop(0, _CH, row, 0)
                    pltpu.async_copy(ob[par], out_h.at[pl.ds(boff + ch * _CH, _CH)],
                                     osem[par])

                    @pl.when(ch + 2 < nch)
                    def _prefetch_next():
                        fire(ch + 2, par)
                return c1

            lax.fori_loop(0, nch // 2, pair, 0)
            return carry

        lax.fori_loop(0, nblk, blk, 0)
        owait(0)
        owait(1)

    return k(did, iid, pid, pd, pp, pi)


def kernel(dataset_ids, image_ids, patch_ids, W_dataset, W_image, W_patch,
           W_proj, b_proj):
    b, l = dataset_ids.shape
    did = dataset_ids.reshape(-1).astype(jnp.int32)
    iid = image_ids.reshape(-1).astype(jnp.int32)
    pid = patch_ids.reshape(-1).astype(jnp.int32)
    e = EMBED_DIM
    pd = _tc_preproject(W_dataset, W_proj[0:e], bias=b_proj, replicas=_REP_D)
    pi = _tc_preproject(W_image, W_proj[e:2 * e], block_rows=20000)
    pp = _tc_preproject(W_patch, W_proj[2 * e:3 * e], replicas=_REP_P)
    out = _sc_gather_sum(did, iid, pid, pd, pp, pi)
    return out.reshape(b, l, MODEL_DIM)


# speedup vs baseline: 1.0308x; 1.0057x over previous
import functools

import jax
import jax.numpy as jnp
from jax import lax
from jax.experimental import pallas as pl
from jax.experimental.pallas import tpu as pltpu
from jax.experimental.pallas import tpu_sc as plsc

EMBED_DIM = 64
MODEL_DIM = 128
_CH = 80
_LANES = 16
_REP_D = 16
_REP_P = 8


def _tc_preproject(tbl, w, bias=None, block_rows=None, replicas=1):
    n = tbl.shape[0]
    br = block_rows or n

    if bias is None:
        def body(t_ref, w_ref, o_ref):
            o_ref[...] = jnp.dot(t_ref[...], w_ref[...],
                                 preferred_element_type=jnp.float32)
        extra_in, extra_spec = (), ()
    else:
        def body(t_ref, w_ref, b_ref, o_ref):
            o_ref[...] = jnp.dot(t_ref[...], w_ref[...],
                                 preferred_element_type=jnp.float32) + b_ref[...]
        extra_in = (bias.reshape(1, MODEL_DIM),)
        extra_spec = (pl.BlockSpec((1, MODEL_DIM), lambda i: (0, 0)),)

    nblocks = n // br
    return pl.pallas_call(
        body,
        grid=(replicas * nblocks,),
        in_specs=[
            pl.BlockSpec((br, EMBED_DIM), lambda i: (i % nblocks, 0)),
            pl.BlockSpec((EMBED_DIM, MODEL_DIM), lambda i: (0, 0)),
            *extra_spec,
        ],
        out_specs=pl.BlockSpec((br, MODEL_DIM), lambda i: (i, 0)),
        out_shape=jax.ShapeDtypeStruct((replicas * n, MODEL_DIM), jnp.float32),
    )(tbl, w, *extra_in)


def _sc_gather_sum(did, iid, pid, pd, pp, pi):
    info = plsc.get_sparse_core_info()
    nc, ns = info.num_cores, info.num_subcores
    nw = nc * ns
    tok = did.shape[0]
    per_w = tok // nw
    idb = 12800
    nblk = per_w // idb
    nch = idb // _CH
    nd, np_ = pd.shape[0], pp.shape[0]
    nd_base, np_base = nd // _REP_D, np_ // _REP_P

    @functools.partial(
        pl.kernel,
        mesh=plsc.VectorSubcoreMesh(core_axis_name="c", subcore_axis_name="s"),
        out_type=jax.ShapeDtypeStruct((tok, MODEL_DIM), jnp.float32),
        scratch_types=[
            pltpu.VMEM((idb,), jnp.int32),
            pltpu.VMEM((idb,), jnp.int32),
            pltpu.VMEM((idb,), jnp.int32),
            [pltpu.VMEM((_CH, MODEL_DIM), jnp.float32) for _ in range(2)],
            [pltpu.VMEM((_CH, MODEL_DIM), jnp.float32) for _ in range(2)],
            [pltpu.VMEM((_CH, MODEL_DIM), jnp.float32) for _ in range(2)],
            [pltpu.VMEM((_CH, MODEL_DIM), jnp.float32) for _ in range(2)],
            [pltpu.SemaphoreType.DMA for _ in range(2)],
            [pltpu.SemaphoreType.DMA for _ in range(2)],
        ],
    )
    def k(did_h, iid_h, pid_h, pd_h, pp_h, pi_h, out_h,
          xd, xi, xp, gd, gi, gp, ob, gsem, osem):
        sid = lax.axis_index("s")
        wid = sid * nc + lax.axis_index("c")
        base = wid * per_w

        if _REP_D > 1 or _REP_P > 1:
            lane = lax.iota(jnp.int32, _LANES)
            spread_d = (lane % _REP_D) * nd_base
            spread_p = (lane % _REP_P) * np_base

        def fire(ch, par):
            off = ch * _CH
            pltpu.async_copy(pi_h.at[xi.at[pl.ds(off, _CH)]], gi[par], gsem[par])
            pltpu.async_copy(pd_h.at[xd.at[pl.ds(off, _CH)]], gd[par], gsem[par])
            pltpu.async_copy(pp_h.at[xp.at[pl.ds(off, _CH)]], gp[par], gsem[par])

        def gwait(par):
            pltpu.make_async_copy(pi_h.at[pl.ds(0, _CH)], gi[par], gsem[par]).wait()
            pltpu.make_async_copy(pi_h.at[pl.ds(0, _CH)], gd[par], gsem[par]).wait()
            pltpu.make_async_copy(pi_h.at[pl.ds(0, _CH)], gp[par], gsem[par]).wait()

        def owait(par):
            pltpu.make_async_copy(pi_h.at[pl.ds(0, _CH)], ob[par], osem[par]).wait()

        def blk(bi_, carry):
            boff = base + bi_ * idb
            pltpu.sync_copy(did_h.at[pl.ds(boff, idb)], xd)
            pltpu.sync_copy(iid_h.at[pl.ds(boff, idb)], xi)
            pltpu.sync_copy(pid_h.at[pl.ds(boff, idb)], xp)

            if _REP_D > 1 or _REP_P > 1:
                def spread(j, c0):
                    s = pl.ds(j * _LANES, _LANES)
                    xd[s] = xd[s] + spread_d
                    xp[s] = xp[s] + spread_p
                    return c0

                lax.fori_loop(0, idb // _LANES, spread, 0)

            fire(0, 0)
            fire(1, 1)

            def pair(pr, c1):
                for par in (0, 1):
                    ch = pr * 2 + par
                    g = bi_ * nch + ch
                    gwait(par)

                    @pl.when(g >= 2)
                    def _drain_prev_write():
                        owait(par)

                    def row(j, c2):
                        for kk in range(MODEL_DIM // _LANES):
                            s = pl.ds(kk * _LANES, _LANES)
                            ob[par][j, s] = gi[par][j, s] + gd[par][j, s] + gp[par][j, s]
                        return c2

                    lax.fori_loop(0, _CH, row, 0)
                    pltpu.async_copy(ob[par], out_h.at[pl.ds(boff + ch * _CH, _CH)],
                                     osem[par])

                    @pl.when(ch + 2 < nch)
                    def _prefetch_next():
                        fire(ch + 2, par)
                return c1

            lax.fori_loop(0, nch // 2, pair, 0)
            return carry

        lax.fori_loop(0, nblk, blk, 0)
        owait(0)
        owait(1)

    return k(did, iid, pid, pd, pp, pi)


def kernel(dataset_ids, image_ids, patch_ids, W_dataset, W_image, W_patch,
           W_proj, b_proj):
    b, l = dataset_ids.shape
    did = dataset_ids.reshape(-1).astype(jnp.int32)
    iid = image_ids.reshape(-1).astype(jnp.int32)
    pid = patch_ids.reshape(-1).astype(jnp.int32)
    e = EMBED_DIM
    pd = _tc_preproject(W_dataset, W_proj[0:e], bias=b_proj, replicas=_REP_D)
    pi = _tc_preproject(W_image, W_proj[e:2 * e], block_rows=25000)
    pp = _tc_preproject(W_patch, W_proj[2 * e:3 * e], replicas=_REP_P)
    out = _sc_gather_sum(did, iid, pid, pd, pp, pi)
    return out.reshape(b, l, MODEL_DIM)
